# baseline (speedup 1.0000x reference)
import jax
import jax.numpy as jnp
from jax import lax
from jax.experimental import pallas as pl
from jax.experimental.pallas import tpu as pltpu

TOKENS = 32768
DIM = 768
NUM_EXPERTS = 64
TOP_K = 2
BLOCK = 4096
HALF = BLOCK // 2


def _top2_scatter(logits):
    iota = lax.broadcasted_iota(jnp.int32, logits.shape, 1).astype(jnp.float32)
    neg_inf = jnp.float32(-jnp.inf)
    big = jnp.float32(NUM_EXPERTS)

    v1 = jnp.max(logits, axis=1, keepdims=True)
    i1 = jnp.min(jnp.where(logits == v1, iota, big), axis=1, keepdims=True)
    hit1 = iota == i1
    masked = jnp.where(hit1, neg_inf, logits)
    v2 = jnp.max(masked, axis=1, keepdims=True)
    i2 = jnp.min(jnp.where(masked == v2, iota, big), axis=1, keepdims=True)
    hit2 = iota == i2

    e2 = jnp.exp(v2 - v1)
    denom = 1.0 + e2
    p1 = 1.0 / denom
    p2 = e2 / denom

    dense = jnp.where(hit1, p1, jnp.where(hit2, p2, 0.0))
    return dense, i1, i2


def _gate_block(x2_ref, w_ref, b_ref, out_ref, idx_ref):
    x2 = x2_ref[...]
    w = w_ref[...]
    bias = b_ref[...]

    nt = (((1,), (1,)), ((), ()))
    logits_l = lax.dot_general(x2[:, :DIM], w, nt,
                               preferred_element_type=jnp.float32) + bias
    logits_r = lax.dot_general(x2[:, DIM:], w, nt,
                               preferred_element_type=jnp.float32) + bias

    dense_l, i1l, i2l = _top2_scatter(logits_l)
    dense_r, i1r, i2r = _top2_scatter(logits_r)

    out_ref[...] = jnp.concatenate([dense_l, dense_r], axis=1)
    quad = jnp.concatenate([i1l, i1r, i2l, i2r], axis=1).astype(jnp.int32)
    idx_ref[...] = quad.T


def _gate(x, W, b):
    x2 = x.reshape(TOKENS // 2, 2 * DIM)
    b2 = b.reshape(1, NUM_EXPERTS)
    grid = (TOKENS // BLOCK,)
    out, idx_q = pl.pallas_call(
        _gate_block,
        grid=grid,
        in_specs=[
            pl.BlockSpec((HALF, 2 * DIM), lambda i: (i, 0)),
            pl.BlockSpec((NUM_EXPERTS, DIM), lambda i: (0, 0)),
            pl.BlockSpec((1, NUM_EXPERTS), lambda i: (0, 0)),
        ],
        out_specs=[
            pl.BlockSpec((HALF, 2 * NUM_EXPERTS), lambda i: (i, 0)),
            pl.BlockSpec((4, HALF), lambda i: (0, i)),
        ],
        out_shape=[
            jax.ShapeDtypeStruct((TOKENS // 2, 2 * NUM_EXPERTS), jnp.float32),
            jax.ShapeDtypeStruct((4, TOKENS // 2), jnp.int32),
        ],
        compiler_params=pltpu.CompilerParams(
            dimension_semantics=("parallel",),
        ),
    )(x2, W, b2)
    dense = out.reshape(TOKENS, NUM_EXPERTS)
    i1 = idx_q[0:2].T.reshape(TOKENS)
    i2 = idx_q[2:4].T.reshape(TOKENS)
    return (dense, jnp.stack([i1, i2], axis=1))


kernel = jax.jit(_gate)


# speedup vs baseline: 2.1671x; 2.1671x over previous
import jax
import jax.numpy as jnp
from jax import lax
from jax.experimental import pallas as pl
from jax.experimental.pallas import tpu as pltpu

TOKENS = 32768
DIM = 768
NUM_EXPERTS = 64
TOP_K = 2
BLOCK = 4096


def _gate_block(x_ref, w_ref, b_ref, out_ref, idx_ref):
    xb = x_ref[...]
    logits = lax.dot_general(
        xb, w_ref[...], (((1,), (1,)), ((), ())),
        preferred_element_type=jnp.float32,
    )
    logits = logits + b_ref[...]

    iota = lax.broadcasted_iota(jnp.int32, logits.shape, 1).astype(jnp.float32)
    neg_inf = jnp.float32(-jnp.inf)
    big = jnp.float32(NUM_EXPERTS)

    v1 = jnp.max(logits, axis=1, keepdims=True)
    i1 = jnp.min(jnp.where(logits == v1, iota, big), axis=1, keepdims=True)
    hit1 = iota == i1
    masked = jnp.where(hit1, neg_inf, logits)
    v2 = jnp.max(masked, axis=1, keepdims=True)
    i2 = jnp.min(jnp.where(masked == v2, iota, big), axis=1, keepdims=True)
    hit2 = iota == i2

    e2 = jnp.exp(v2 - v1)
    denom = 1.0 + e2
    p1 = 1.0 / denom
    p2 = e2 / denom

    dense = jnp.where(hit1, p1, jnp.where(hit2, p2, 0.0))
    half = dense.shape[0] // 2
    out_ref[...] = jnp.concatenate([dense[:half, :], dense[half:, :]], axis=1)

    pair = jnp.concatenate([i1, i2], axis=1).astype(jnp.int32)
    idx_ref[...] = pair.T


def _gate(x, W, b):
    b2 = b.reshape(1, NUM_EXPERTS)
    grid = (TOKENS // BLOCK,)
    out, idx_t = pl.pallas_call(
        _gate_block,
        grid=grid,
        in_specs=[
            pl.BlockSpec((BLOCK, DIM), lambda i: (i, 0)),
            pl.BlockSpec((NUM_EXPERTS, DIM), lambda i: (0, 0)),
            pl.BlockSpec((1, NUM_EXPERTS), lambda i: (0, 0)),
        ],
        out_specs=[
            pl.BlockSpec((BLOCK // 2, 2 * NUM_EXPERTS), lambda i: (i, 0)),
            pl.BlockSpec((TOP_K, BLOCK), lambda i: (0, i)),
        ],
        out_shape=[
            jax.ShapeDtypeStruct((TOKENS // 2, 2 * NUM_EXPERTS), jnp.float32),
            jax.ShapeDtypeStruct((TOP_K, TOKENS), jnp.int32),
        ],
        compiler_params=pltpu.CompilerParams(
            dimension_semantics=("parallel",),
        ),
    )(x, W, b2)
    nb = TOKENS // BLOCK
    dense = (
        out.reshape(nb, BLOCK // 2, 2, NUM_EXPERTS)
        .swapaxes(1, 2)
        .reshape(TOKENS, NUM_EXPERTS)
    )
    return (dense, idx_t.T)


kernel = jax.jit(_gate)


# speedup vs baseline: 3.9436x; 1.8198x over previous
import jax
import jax.numpy as jnp
from jax import lax
from jax.experimental import pallas as pl
from jax.experimental.pallas import tpu as pltpu
from jax.experimental.layout import Format, Layout

TOKENS = 32768
DIM = 768
NUM_EXPERTS = 64
TOP_K = 2
BLOCK = 4096


def _gate_block(x_ref, w_ref, b_ref, out_ref, idx_ref):
    xb = x_ref[...]
    logits = lax.dot_general(
        xb, w_ref[...], (((1,), (1,)), ((), ())),
        preferred_element_type=jnp.float32,
    )
    logits = logits + b_ref[...]

    iota = lax.broadcasted_iota(jnp.int32, logits.shape, 1).astype(jnp.float32)
    neg_inf = jnp.float32(-jnp.inf)
    big = jnp.float32(NUM_EXPERTS)

    v1 = jnp.max(logits, axis=1, keepdims=True)
    i1 = jnp.min(jnp.where(logits == v1, iota, big), axis=1, keepdims=True)
    hit1 = iota == i1
    masked = jnp.where(hit1, neg_inf, logits)
    v2 = jnp.max(masked, axis=1, keepdims=True)
    i2 = jnp.min(jnp.where(masked == v2, iota, big), axis=1, keepdims=True)
    hit2 = iota == i2

    e2 = jnp.exp(v2 - v1)
    denom = 1.0 + e2
    p1 = 1.0 / denom
    p2 = e2 / denom

    out_ref[...] = jnp.where(hit1, p1, jnp.where(hit2, p2, 0.0))
    pair = jnp.concatenate([i1, i2], axis=1).astype(jnp.int32)
    idx_ref[...] = pair.T


def _gate(x, W, b):
    b2 = b.reshape(1, NUM_EXPERTS)
    grid = (TOKENS // BLOCK,)
    out, idx_t = pl.pallas_call(
        _gate_block,
        grid=grid,
        in_specs=[
            pl.BlockSpec((BLOCK, DIM), lambda i: (i, 0)),
            pl.BlockSpec((NUM_EXPERTS, DIM), lambda i: (0, 0)),
            pl.BlockSpec((1, NUM_EXPERTS), lambda i: (0, 0)),
        ],
        out_specs=[
            pl.BlockSpec((BLOCK, NUM_EXPERTS), lambda i: (i, 0)),
            pl.BlockSpec((TOP_K, BLOCK), lambda i: (0, i)),
        ],
        out_shape=[
            jax.ShapeDtypeStruct((TOKENS, NUM_EXPERTS), jnp.float32),
            jax.ShapeDtypeStruct((TOP_K, TOKENS), jnp.int32),
        ],
        compiler_params=pltpu.CompilerParams(
            dimension_semantics=("parallel",),
        ),
    )(x, W, b2)
    return (out, idx_t.T)


kernel = jax.jit(_gate)


# speedup vs baseline: 5.5713x; 1.4128x over previous
import jax
import jax.numpy as jnp
from jax import lax
from jax.experimental import pallas as pl
from jax.experimental.pallas import tpu as pltpu

TOKENS = 32768
DIM = 768
NUM_EXPERTS = 64
TOP_K = 2
BLOCK = 4096


def _gate_block(x_ref, w_ref, b_ref, out_ref, idx_ref):
    xb = x_ref[...]
    logits_t = lax.dot_general(
        w_ref[...], xb, (((1,), (1,)), ((), ())),
        preferred_element_type=jnp.float32,
    )
    logits_t = logits_t + b_ref[...]

    iota = lax.broadcasted_iota(jnp.int32, logits_t.shape, 0).astype(jnp.float32)
    neg_inf = jnp.float32(-jnp.inf)
    big = jnp.float32(NUM_EXPERTS)

    v1 = jnp.max(logits_t, axis=0, keepdims=True)
    i1 = jnp.min(jnp.where(logits_t == v1, iota, big), axis=0, keepdims=True)
    hit1 = iota == i1
    masked = jnp.where(hit1, neg_inf, logits_t)
    v2 = jnp.max(masked, axis=0, keepdims=True)
    i2 = jnp.min(jnp.where(masked == v2, iota, big), axis=0, keepdims=True)
    hit2 = iota == i2

    e2 = jnp.exp(v2 - v1)
    denom = 1.0 + e2
    p1 = 1.0 / denom
    p2 = e2 / denom

    out_ref[...] = jnp.where(hit1, p1, jnp.where(hit2, p2, 0.0))
    idx_ref[...] = jnp.concatenate([i1, i2], axis=0).astype(jnp.int32)


def _gate(x, W, b):
    b_col = b.reshape(NUM_EXPERTS, 1)
    grid = (TOKENS // BLOCK,)
    out_t, idx_t = pl.pallas_call(
        _gate_block,
        grid=grid,
        in_specs=[
            pl.BlockSpec((BLOCK, DIM), lambda i: (i, 0)),
            pl.BlockSpec((NUM_EXPERTS, DIM), lambda i: (0, 0)),
            pl.BlockSpec((NUM_EXPERTS, 1), lambda i: (0, 0)),
        ],
        out_specs=[
            pl.BlockSpec((NUM_EXPERTS, BLOCK), lambda i: (0, i)),
            pl.BlockSpec((TOP_K, BLOCK), lambda i: (0, i)),
        ],
        out_shape=[
            jax.ShapeDtypeStruct((NUM_EXPERTS, TOKENS), jnp.float32),
            jax.ShapeDtypeStruct((TOP_K, TOKENS), jnp.int32),
        ],
        compiler_params=pltpu.CompilerParams(
            dimension_semantics=("parallel",),
        ),
    )(x, W, b_col)
    return (out_t.T, idx_t.T)


kernel = jax.jit(_gate)
